# baseline (speedup 1.0000x reference)
import functools

import jax
import jax.numpy as jnp
from jax import lax
from jax.experimental import pallas as pl
from jax.experimental.pallas import tpu as pltpu

BATCH = 4096
EMB = 4096
DIM = 8192
BR = 512


def _tc_body(idx_ref, dv_ref, x_ref, w_ref, b_ref, cache_ref, out_ref):
    idx = idx_ref[0]
    iota = lax.broadcasted_iota(jnp.int32, (1, DIM), 1)
    sel = (iota == idx).astype(jnp.float32)
    w = jnp.sum(w_ref[...] * sel)
    b = jnp.sum(b_ref[...] * sel)
    dv = jnp.clip(dv_ref[0], 0.9, 1.0)
    addend = dv * cache_ref[...] + b
    out_ref[...] = x_ref[...] * w + addend


BC = 2048

_grid_call = pl.pallas_call(
    _tc_body,
    grid=(BATCH // BR, EMB // BC),
    in_specs=[
        pl.BlockSpec(memory_space=pltpu.SMEM),
        pl.BlockSpec(memory_space=pltpu.SMEM),
        pl.BlockSpec((BR, BC), lambda i, j: (i, j)),
        pl.BlockSpec((1, DIM), lambda i, j: (0, 0)),
        pl.BlockSpec((1, DIM), lambda i, j: (0, 0)),
        pl.BlockSpec((1, BC), lambda i, j: (0, j)),
    ],
    out_specs=pl.BlockSpec((BR, BC), lambda i, j: (i, j)),
    out_shape=jax.ShapeDtypeStruct((BATCH, EMB), jnp.float32),
)


@jax.jit
def kernel(x, index, weight, bias, decay_value, cache):
    idx1 = jnp.asarray(index, jnp.int32).reshape(1)
    dv1 = decay_value.astype(jnp.float32).reshape(1)
    return _grid_call(idx1, dv1, x, weight.reshape(1, DIM),
                      bias.reshape(1, DIM), cache.reshape(1, EMB))


# speedup vs baseline: 1.0597x; 1.0597x over previous
import jax
import jax.numpy as jnp
from jax import lax
from jax.experimental import pallas as pl
from jax.experimental.pallas import tpu as pltpu

BATCH = 4096
EMB = 4096
DIM = 8192
CH = 256
NCH = BATCH // CH
NB = 4


def _tc_body(idx_ref, dv_ref, x_any, w_ref, b_ref, cache_ref, out_any,
             b0, b1, b2, b3, si0, si1, si2, si3, so0, so1, so2, so3):
    bufs = (b0, b1, b2, b3)
    in_sems = (si0, si1, si2, si3)
    out_sems = (so0, so1, so2, so3)

    idx = idx_ref[0]
    iota = lax.broadcasted_iota(jnp.int32, (1, DIM), 1)
    sel = (iota == idx).astype(jnp.float32)
    w = jnp.sum(w_ref[...] * sel)
    b = jnp.sum(b_ref[...] * sel)
    dv = jnp.clip(dv_ref[0], 0.9, 1.0)
    addend = dv * cache_ref[...] + b

    def start_in(g):
        return pltpu.make_async_copy(
            x_any.at[pl.ds(g * CH, CH), :], bufs[g % NB], in_sems[g % NB])

    def start_out(g):
        return pltpu.make_async_copy(
            bufs[g % NB], out_any.at[pl.ds(g * CH, CH), :], out_sems[g % NB])

    for g in range(min(3, NCH)):
        start_in(g).start()
    outs = {}
    for g in range(NCH):
        start_in(g).wait()
        buf = bufs[g % NB]
        buf[...] = buf[...] * w + addend
        outs[g] = start_out(g)
        outs[g].start()
        if g + 3 < NCH:
            if g >= 1:
                outs[g - 1].wait()
            start_in(g + 3).start()
    for g in range(max(0, NCH - 4), NCH):
        outs[g].wait()


_call = pl.pallas_call(
    _tc_body,
    in_specs=[
        pl.BlockSpec(memory_space=pltpu.SMEM),
        pl.BlockSpec(memory_space=pltpu.SMEM),
        pl.BlockSpec(memory_space=pl.ANY),
        pl.BlockSpec(memory_space=pltpu.VMEM),
        pl.BlockSpec(memory_space=pltpu.VMEM),
        pl.BlockSpec(memory_space=pltpu.VMEM),
    ],
    out_specs=pl.BlockSpec(memory_space=pl.ANY),
    out_shape=jax.ShapeDtypeStruct((BATCH, EMB), jnp.float32),
    scratch_shapes=(
        [pltpu.VMEM((CH, EMB), jnp.float32)] * NB
        + [pltpu.SemaphoreType.DMA] * (2 * NB)
    ),
)


@jax.jit
def kernel(x, index, weight, bias, decay_value, cache):
    idx1 = jnp.asarray(index, jnp.int32).reshape(1)
    dv1 = decay_value.astype(jnp.float32).reshape(1)
    return _call(idx1, dv1, x, weight.reshape(1, DIM),
                 bias.reshape(1, DIM), cache.reshape(1, EMB))


# speedup vs baseline: 1.0675x; 1.0074x over previous
import jax
import jax.numpy as jnp
from jax import lax
from jax.experimental import pallas as pl
from jax.experimental.pallas import tpu as pltpu

BATCH = 4096
EMB = 4096
DIM = 8192
CH = 512
NCH = BATCH // CH
NB = 4


def _tc_body(idx_ref, dv_ref, x_any, w_ref, b_ref, cache_ref, out_any,
             b0, b1, b2, b3, si0, si1, si2, si3, so0, so1, so2, so3):
    bufs = (b0, b1, b2, b3)
    in_sems = (si0, si1, si2, si3)
    out_sems = (so0, so1, so2, so3)

    idx = idx_ref[0]
    iota = lax.broadcasted_iota(jnp.int32, (1, DIM), 1)
    sel = (iota == idx).astype(jnp.float32)
    w = jnp.sum(w_ref[...] * sel)
    b = jnp.sum(b_ref[...] * sel)
    dv = jnp.clip(dv_ref[0], 0.9, 1.0)
    addend = dv * cache_ref[...] + b

    def start_in(g):
        return pltpu.make_async_copy(
            x_any.at[pl.ds(g * CH, CH), :], bufs[g % NB], in_sems[g % NB])

    def start_out(g):
        return pltpu.make_async_copy(
            bufs[g % NB], out_any.at[pl.ds(g * CH, CH), :], out_sems[g % NB])

    for g in range(min(3, NCH)):
        start_in(g).start()
    outs = {}
    for g in range(NCH):
        start_in(g).wait()
        buf = bufs[g % NB]
        buf[...] = buf[...] * w + addend
        outs[g] = start_out(g)
        outs[g].start()
        if g + 3 < NCH:
            if g >= 1:
                outs[g - 1].wait()
            start_in(g + 3).start()
    for g in range(max(0, NCH - 4), NCH):
        outs[g].wait()


_call = pl.pallas_call(
    _tc_body,
    in_specs=[
        pl.BlockSpec(memory_space=pltpu.SMEM),
        pl.BlockSpec(memory_space=pltpu.SMEM),
        pl.BlockSpec(memory_space=pl.ANY),
        pl.BlockSpec(memory_space=pltpu.VMEM),
        pl.BlockSpec(memory_space=pltpu.VMEM),
        pl.BlockSpec(memory_space=pltpu.VMEM),
    ],
    out_specs=pl.BlockSpec(memory_space=pl.ANY),
    out_shape=jax.ShapeDtypeStruct((BATCH, EMB), jnp.float32),
    scratch_shapes=(
        [pltpu.VMEM((CH, EMB), jnp.float32)] * NB
        + [pltpu.SemaphoreType.DMA] * (2 * NB)
    ),
)


@jax.jit
def kernel(x, index, weight, bias, decay_value, cache):
    idx1 = jnp.asarray(index, jnp.int32).reshape(1)
    dv1 = decay_value.astype(jnp.float32).reshape(1)
    return _call(idx1, dv1, x, weight.reshape(1, DIM),
                 bias.reshape(1, DIM), cache.reshape(1, EMB))
